# baseline (speedup 1.0000x reference)
import functools
import math

import jax
import jax.numpy as jnp
from jax import lax
from jax.experimental import pallas as pl
from jax.experimental.pallas import tpu as pltpu
from jax.experimental.pallas import tpu_sc as plsc

_BN_EPS = 1e-3
_B, _N, _E, _D, _T, _H = 2, 10000, 320000, 128, 32, 128
_NC, _NS = 2, 16
_EP = _E // _NS
_CH = 80
_NCHUNK = _EP // _CH
_NP = 10240
_RPS = _NP // _NS
_ZR = 40
_E2 = _E // _NC
_EP2 = _E2 // _NS
_INV_SQRT2 = 0.7071067811865476

_f32 = jnp.float32


def _gelu(x):
    return 0.5 * x * (1.0 + lax.erf(x * _INV_SQRT2))


def _msg_body(x_ref, w_ref, c_ref, o_ref):
    h = jnp.dot(x_ref[0], w_ref[...], preferred_element_type=_f32)
    o_ref[0] = _gelu(h + c_ref[0])


def _msg_table(x, w1x, c1):
    blk = 2000
    return pl.pallas_call(
        _msg_body,
        grid=(_B, _N // blk),
        in_specs=[
            pl.BlockSpec((1, blk, _D), lambda b, i: (b, i, 0)),
            pl.BlockSpec((_D, _H), lambda b, i: (0, 0)),
            pl.BlockSpec((1, 1, _H), lambda b, i: (b, 0, 0)),
        ],
        out_specs=pl.BlockSpec((1, blk, _H), lambda b, i: (b, i, 0)),
        out_shape=jax.ShapeDtypeStruct((_B, _N, _H), _f32),
    )(x, w1x, c1)


def _sc_sums_body(m_hbm, nbr_hbm, sums_hbm,
                  idx0, idx1, rows0, rows1, ztile, acc_sh,
                  sem0, sem1):
    b = lax.axis_index("c")
    s = lax.axis_index("s")

    z16 = jnp.zeros((16,), _f32)
    for j in range(_ZR):
        for k in range(_H // 16):
            ztile[j, pl.ds(k * 16, 16)] = z16

    @pl.loop(0, _RPS // _ZR)
    def _zinit(i):
        pltpu.sync_copy(ztile, acc_sh.at[pl.ds(s * _RPS + i * _ZR, _ZR)])

    plsc.subcore_barrier()

    bufs = ((idx0, rows0, sem0), (idx1, rows1, sem1))

    for t in range(2):
        pk_v, rows_v, sem = bufs[t]
        pltpu.sync_copy(nbr_hbm.at[b, s, t], pk_v)
        pltpu.async_copy(m_hbm.at[pk_v.at[0]], rows_v, sem)

    @pl.loop(0, _NCHUNK, step=2)
    def _step(i):
        for t in range(2):
            pk_v, rows_v, sem = bufs[t]
            j = i + t
            pltpu.make_async_copy(m_hbm.at[pk_v.at[0]], rows_v, sem).wait()
            pltpu.sync_copy(rows_v, acc_sh.at[pk_v.at[1]], add=True)
            pltpu.sync_copy(nbr_hbm.at[b, s, j + 2], pk_v)
            pltpu.async_copy(m_hbm.at[pk_v.at[0]], rows_v, sem)

    for t in range(2):
        pk_v, rows_v, sem = bufs[t]
        pltpu.make_async_copy(m_hbm.at[pk_v.at[0]], rows_v, sem).wait()

    plsc.subcore_barrier()

    @pl.loop(0, _RPS // _ZR)
    def _wb(i):
        r0 = s * _RPS + i * _ZR
        pltpu.sync_copy(acc_sh.at[pl.ds(r0, _ZR)], ztile)
        pltpu.sync_copy(ztile, sums_hbm.at[b, pl.ds(r0, _ZR)])


def _sc_cnt_body(seg_hbm, cnt_hbm, seg_v, ones_v, ztile, cnt_sh):
    c = lax.axis_index("c")
    s = lax.axis_index("s")

    z16 = jnp.zeros((16,), _f32)
    one16 = jnp.ones((16,), _f32)
    for j in range(_ZR):
        for k in range(_H // 16):
            ztile[j, pl.ds(k * 16, 16)] = z16
    for j in range(_CH):
        for k in range(_H // 16):
            ones_v[j, pl.ds(k * 16, 16)] = one16

    @pl.loop(0, _RPS // _ZR)
    def _zinit(i):
        pltpu.sync_copy(ztile, cnt_sh.at[pl.ds(s * _RPS + i * _ZR, _ZR)])

    plsc.subcore_barrier()

    ebase = c * _E2 + s * _EP2

    @pl.loop(0, _EP2 // _CH)
    def _step(i):
        pltpu.sync_copy(seg_hbm.at[pl.ds(ebase + i * _CH, _CH)], seg_v)
        pltpu.sync_copy(ones_v, cnt_sh.at[seg_v], add=True)

    plsc.subcore_barrier()

    @pl.loop(0, _RPS // _ZR)
    def _wb(i):
        r0 = s * _RPS + i * _ZR
        pltpu.sync_copy(cnt_sh.at[pl.ds(r0, _ZR)], ztile)
        pltpu.sync_copy(ztile, cnt_hbm.at[c, pl.ds(r0, _ZR)])


def _sc_aggregate(m_flat, pk, seg):
    mesh = plsc.VectorSubcoreMesh(core_axis_name="c", subcore_axis_name="s",
                                  num_cores=_NC, num_subcores=_NS)
    ksum = functools.partial(
        pl.kernel,
        out_type=jax.ShapeDtypeStruct((_B, _NP, _H), _f32),
        mesh=mesh,
        scratch_types=[
            pltpu.VMEM((2, _CH), jnp.int32),
            pltpu.VMEM((2, _CH), jnp.int32),
            pltpu.VMEM((_CH, _H), _f32),
            pltpu.VMEM((_CH, _H), _f32),
            pltpu.VMEM((_ZR, _H), _f32),
            pltpu.VMEM_SHARED((_NP, _H), _f32),
            pltpu.SemaphoreType.DMA,
            pltpu.SemaphoreType.DMA,
        ],
    )(_sc_sums_body)
    kcnt = functools.partial(
        pl.kernel,
        out_type=jax.ShapeDtypeStruct((_NC, _NP, _H), _f32),
        mesh=mesh,
        scratch_types=[
            pltpu.VMEM((_CH,), jnp.int32),
            pltpu.VMEM((_CH, _H), _f32),
            pltpu.VMEM((_ZR, _H), _f32),
            pltpu.VMEM_SHARED((_NP, _H), _f32),
        ],
    )(_sc_cnt_body)
    return ksum(m_flat, pk), kcnt(seg)


def _upd_body(x_ref, s_ref, c2_ref, wx_ref, wh_ref, c_ref, o_ref):
    cnt = c2_ref[0][:, 0:1] + c2_ref[1][:, 0:1]
    agg = s_ref[0] * (1.0 / jnp.maximum(cnt, 1.0))
    h = (jnp.dot(x_ref[0], wx_ref[...], preferred_element_type=_f32)
         + jnp.dot(agg, wh_ref[...], preferred_element_type=_f32))
    o_ref[0] = _gelu(h + c_ref[0])


def _update(x, sums, cnt16, w2x, w2h, c2):
    blk = 2000
    return pl.pallas_call(
        _upd_body,
        grid=(_B, _N // blk),
        in_specs=[
            pl.BlockSpec((1, blk, _D), lambda b, i: (b, i, 0)),
            pl.BlockSpec((1, blk, _H), lambda b, i: (b, i, 0)),
            pl.BlockSpec((_NC, blk, _H), lambda b, i: (0, i, 0)),
            pl.BlockSpec((_D, _H), lambda b, i: (0, 0)),
            pl.BlockSpec((_H, _H), lambda b, i: (0, 0)),
            pl.BlockSpec((1, 1, _H), lambda b, i: (b, 0, 0)),
        ],
        out_specs=pl.BlockSpec((1, blk, _H), lambda b, i: (b, i, 0)),
        out_shape=jax.ShapeDtypeStruct((_B, _N, _H), _f32),
    )(x, sums, cnt16, w2x, w2h, c2)


def kernel(node_representations, edges, edge_weights, time_embed,
           gamma1, beta1, W1, b1, gamma2, beta2, W2, b2):
    x = node_representations
    inv = (1.0 + _BN_EPS) ** -0.5
    s1 = gamma1 * inv
    s2 = gamma2 * inv

    w1x = W1[:_D] * s1[:_D, None]
    c1 = (beta1 @ W1 + (time_embed * s1[_D:]) @ W1[_D:] + b1)[:, None, :]
    w2x = W2[:_D] * s2[:_D, None]
    w2h = W2[_D:_D + _H] * s2[_D:_D + _H, None]
    c2 = (beta2 @ W2 + (time_embed * s2[_D + _H:]) @ W2[_D + _H:] + b2)[:, None, :]

    msgs = _msg_table(x, w1x, c1)
    m_flat = msgs.reshape(_B * _N, _H)

    nbr = edges[:, :, 1].astype(jnp.int32)
    nbr_off = nbr + jnp.arange(_B, dtype=jnp.int32)[:, None] * _N
    seg = edges[0, :, 0].astype(jnp.int32)
    nbr_r = nbr_off.reshape(_B, _NS, _NCHUNK, _CH)
    seg_r = jnp.broadcast_to(seg.reshape(1, _NS, _NCHUNK, _CH),
                             (_B, _NS, _NCHUNK, _CH))
    pk = jnp.stack([nbr_r, seg_r], axis=3)
    pk = jnp.pad(pk, ((0, 0), (0, 0), (0, 2), (0, 0), (0, 0)))

    sums, cnt16 = _sc_aggregate(m_flat, pk, seg)

    return _update(x, sums, cnt16, w2x, w2h, c2)


# speedup vs baseline: 1.3725x; 1.3725x over previous
import functools
import math

import jax
import jax.numpy as jnp
from jax import lax
from jax.experimental import pallas as pl
from jax.experimental.pallas import tpu as pltpu
from jax.experimental.pallas import tpu_sc as plsc

_BN_EPS = 1e-3
_B, _N, _E, _D, _T, _H = 2, 10000, 320000, 128, 32, 128
_NC, _NS = 2, 16
_EP = _E // _NS
_CH = 80
_NCHUNK = _EP // _CH
_NP = 10240
_RPS = _NP // _NS
_ZR = 40
_E2 = _E // _NC
_EP2 = _E2 // _NS
_INV_SQRT2 = 0.7071067811865476

_f32 = jnp.float32


def _gelu(x):
    return 0.5 * x * (1.0 + lax.erf(x * _INV_SQRT2))


def _msg_body(x_ref, w_ref, c_ref, o_ref):
    h = jnp.dot(x_ref[0], w_ref[...], preferred_element_type=_f32)
    o_ref[0] = _gelu(h + c_ref[0])


def _msg_table(x, w1x, c1):
    blk = 2000
    return pl.pallas_call(
        _msg_body,
        grid=(_B, _N // blk),
        in_specs=[
            pl.BlockSpec((1, blk, _D), lambda b, i: (b, i, 0)),
            pl.BlockSpec((_D, _H), lambda b, i: (0, 0)),
            pl.BlockSpec((1, 1, _H), lambda b, i: (b, 0, 0)),
        ],
        out_specs=pl.BlockSpec((1, blk, _H), lambda b, i: (b, i, 0)),
        out_shape=jax.ShapeDtypeStruct((_B, _N, _H), _f32),
    )(x, w1x, c1)


def _sc_sums_body(m_hbm, nbr_hbm, seg_hbm, sums_hbm,
                  idx0, idx1, seg0, seg1, rows0, rows1, ztile, acc_sh,
                  sem0, sem1):
    b = lax.axis_index("c")
    s = lax.axis_index("s")

    z16 = jnp.zeros((16,), _f32)
    for j in range(_ZR):
        for k in range(_H // 16):
            ztile[j, pl.ds(k * 16, 16)] = z16

    @pl.loop(0, _RPS // _ZR)
    def _zinit(i):
        pltpu.sync_copy(ztile, acc_sh.at[pl.ds(s * _RPS + i * _ZR, _ZR)])

    plsc.subcore_barrier()

    ebase = b * _E + s * _EP
    bufs = ((idx0, seg0, rows0, sem0), (idx1, seg1, rows1, sem1))

    for t in range(2):
        idx_v, seg_v, rows_v, sem = bufs[t]
        pltpu.sync_copy(nbr_hbm.at[pl.ds(ebase + t * _CH, _CH)], idx_v)
        pltpu.sync_copy(seg_hbm.at[pl.ds(s * _EP + t * _CH, _CH)], seg_v)
        pltpu.async_copy(m_hbm.at[idx_v], rows_v, sem)

    @pl.loop(0, _NCHUNK, step=2)
    def _step(i):
        for t in range(2):
            idx_v, seg_v, rows_v, sem = bufs[t]
            j = i + t
            pltpu.make_async_copy(m_hbm.at[idx_v], rows_v, sem).wait()
            pltpu.sync_copy(rows_v, acc_sh.at[seg_v], add=True)
            off = ebase + (j + 2) * _CH
            pltpu.sync_copy(nbr_hbm.at[pl.ds(off, _CH)], idx_v)
            pltpu.sync_copy(seg_hbm.at[pl.ds(s * _EP + (j + 2) * _CH, _CH)],
                            seg_v)
            pltpu.async_copy(m_hbm.at[idx_v], rows_v, sem)

    for t in range(2):
        idx_v, seg_v, rows_v, sem = bufs[t]
        pltpu.make_async_copy(m_hbm.at[idx_v], rows_v, sem).wait()

    plsc.subcore_barrier()

    @pl.loop(0, _RPS // _ZR)
    def _wb(i):
        r0 = s * _RPS + i * _ZR
        pltpu.sync_copy(acc_sh.at[pl.ds(r0, _ZR)], ztile)
        pltpu.sync_copy(ztile, sums_hbm.at[b, pl.ds(r0, _ZR)])


def _sc_cnt_body(seg_hbm, cnt_hbm, seg0, seg1, ones_v, ztile, cnt_sh,
                 sem0, sem1):
    c = lax.axis_index("c")
    s = lax.axis_index("s")

    z16 = jnp.zeros((16,), _f32)
    one16 = jnp.ones((16,), _f32)
    for j in range(_ZR):
        for k in range(_H // 16):
            ztile[j, pl.ds(k * 16, 16)] = z16
    for j in range(_CH):
        for k in range(_H // 16):
            ones_v[j, pl.ds(k * 16, 16)] = one16

    @pl.loop(0, _RPS // _ZR)
    def _zinit(i):
        pltpu.sync_copy(ztile, cnt_sh.at[pl.ds(s * _RPS + i * _ZR, _ZR)])

    plsc.subcore_barrier()

    ebase = c * _E2 + s * _EP2
    bufs = ((seg0, sem0), (seg1, sem1))
    ncc = _EP2 // _CH

    for t in range(2):
        seg_v, sem = bufs[t]
        pltpu.async_copy(seg_hbm.at[pl.ds(ebase + t * _CH, _CH)], seg_v, sem)

    @pl.loop(0, ncc - 1, step=2)
    def _step(i):
        for t in range(2):
            seg_v, sem = bufs[t]
            j = i + t
            pltpu.make_async_copy(
                seg_hbm.at[pl.ds(ebase, _CH)], seg_v, sem).wait()
            pltpu.sync_copy(ones_v, cnt_sh.at[seg_v], add=True)
            pltpu.async_copy(
                seg_hbm.at[pl.ds(ebase + (j + 2) * _CH, _CH)], seg_v, sem)

    seg_v, sem = bufs[(ncc - 1) % 2]
    pltpu.make_async_copy(seg_hbm.at[pl.ds(ebase, _CH)], seg_v, sem).wait()
    pltpu.sync_copy(ones_v, cnt_sh.at[seg_v], add=True)
    seg_v, sem = bufs[ncc % 2]
    pltpu.make_async_copy(seg_hbm.at[pl.ds(ebase, _CH)], seg_v, sem).wait()

    plsc.subcore_barrier()

    @pl.loop(0, _RPS // _ZR)
    def _wb(i):
        r0 = s * _RPS + i * _ZR
        pltpu.sync_copy(cnt_sh.at[pl.ds(r0, _ZR)], ztile)
        pltpu.sync_copy(ztile, cnt_hbm.at[c, pl.ds(r0, _ZR)])


def _sc_aggregate(m_flat, nbr_flat, seg):
    mesh = plsc.VectorSubcoreMesh(core_axis_name="c", subcore_axis_name="s",
                                  num_cores=_NC, num_subcores=_NS)
    ksum = functools.partial(
        pl.kernel,
        out_type=jax.ShapeDtypeStruct((_B, _NP, _H), _f32),
        mesh=mesh,
        scratch_types=[
            pltpu.VMEM((_CH,), jnp.int32),
            pltpu.VMEM((_CH,), jnp.int32),
            pltpu.VMEM((_CH,), jnp.int32),
            pltpu.VMEM((_CH,), jnp.int32),
            pltpu.VMEM((_CH, _H), _f32),
            pltpu.VMEM((_CH, _H), _f32),
            pltpu.VMEM((_ZR, _H), _f32),
            pltpu.VMEM_SHARED((_NP, _H), _f32),
            pltpu.SemaphoreType.DMA,
            pltpu.SemaphoreType.DMA,
        ],
    )(_sc_sums_body)
    kcnt = functools.partial(
        pl.kernel,
        out_type=jax.ShapeDtypeStruct((_NC, _NP, _H), _f32),
        mesh=mesh,
        scratch_types=[
            pltpu.VMEM((_CH,), jnp.int32),
            pltpu.VMEM((_CH,), jnp.int32),
            pltpu.VMEM((_CH, _H), _f32),
            pltpu.VMEM((_ZR, _H), _f32),
            pltpu.VMEM_SHARED((_NP, _H), _f32),
            pltpu.SemaphoreType.DMA,
            pltpu.SemaphoreType.DMA,
        ],
    )(_sc_cnt_body)
    return ksum(m_flat, nbr_flat, seg), kcnt(seg)


def _upd_body(x_ref, s_ref, c2_ref, wx_ref, wh_ref, c_ref, o_ref):
    cnt = c2_ref[0][:, 0:1] + c2_ref[1][:, 0:1]
    agg = s_ref[0] * (1.0 / jnp.maximum(cnt, 1.0))
    h = (jnp.dot(x_ref[0], wx_ref[...], preferred_element_type=_f32)
         + jnp.dot(agg, wh_ref[...], preferred_element_type=_f32))
    o_ref[0] = _gelu(h + c_ref[0])


def _update(x, sums, cnt16, w2x, w2h, c2):
    blk = 2000
    return pl.pallas_call(
        _upd_body,
        grid=(_B, _N // blk),
        in_specs=[
            pl.BlockSpec((1, blk, _D), lambda b, i: (b, i, 0)),
            pl.BlockSpec((1, blk, _H), lambda b, i: (b, i, 0)),
            pl.BlockSpec((_NC, blk, _H), lambda b, i: (0, i, 0)),
            pl.BlockSpec((_D, _H), lambda b, i: (0, 0)),
            pl.BlockSpec((_H, _H), lambda b, i: (0, 0)),
            pl.BlockSpec((1, 1, _H), lambda b, i: (b, 0, 0)),
        ],
        out_specs=pl.BlockSpec((1, blk, _H), lambda b, i: (b, i, 0)),
        out_shape=jax.ShapeDtypeStruct((_B, _N, _H), _f32),
    )(x, sums, cnt16, w2x, w2h, c2)


def kernel(node_representations, edges, edge_weights, time_embed,
           gamma1, beta1, W1, b1, gamma2, beta2, W2, b2):
    x = node_representations
    inv = (1.0 + _BN_EPS) ** -0.5
    s1 = gamma1 * inv
    s2 = gamma2 * inv

    w1x = W1[:_D] * s1[:_D, None]
    c1 = (beta1 @ W1 + (time_embed * s1[_D:]) @ W1[_D:] + b1)[:, None, :]
    w2x = W2[:_D] * s2[:_D, None]
    w2h = W2[_D:_D + _H] * s2[_D:_D + _H, None]
    c2 = (beta2 @ W2 + (time_embed * s2[_D + _H:]) @ W2[_D + _H:] + b2)[:, None, :]

    msgs = _msg_table(x, w1x, c1)
    m_flat = msgs.reshape(_B * _N, _H)

    nbr = edges[:, :, 1].astype(jnp.int32)
    nbr_flat = (nbr + jnp.arange(_B, dtype=jnp.int32)[:, None] * _N).reshape(-1)
    seg = edges[0, :, 0].astype(jnp.int32)
    nbr_flat = jnp.pad(nbr_flat, (0, 2 * _CH))
    seg = jnp.pad(seg, (0, 2 * _CH))

    sums, cnt16 = _sc_aggregate(m_flat, nbr_flat, seg)

    return _update(x, sums, cnt16, w2x, w2h, c2)


# speedup vs baseline: 1.3735x; 1.0007x over previous
import functools

import jax
import jax.numpy as jnp
from jax import lax
from jax.experimental import pallas as pl
from jax.experimental.pallas import tpu as pltpu
from jax.experimental.pallas import tpu_sc as plsc

_BN_EPS = 1e-3
_B, _N, _E, _D, _T, _H = 2, 10000, 320000, 128, 32, 128
_NC, _NS = 2, 16
_EP = _E // _NS
_CH = 80
_NCHUNK = _EP // _CH
_NP = 10240
_RPS = _NP // _NS
_ZR = 40
_E2 = _E // _NC
_EP2 = _E2 // _NS
_INV_SQRT2 = 0.7071067811865476

_f32 = jnp.float32


def _gelu(x):
    return 0.5 * x * (1.0 + lax.erf(x * _INV_SQRT2))


def _msg_body(x_ref, w_ref, c_ref, o_ref):
    h = jnp.dot(x_ref[0], w_ref[...], preferred_element_type=_f32)
    o_ref[0] = _gelu(h + c_ref[0])


def _msg_table(x, w1x, c1):
    blk = 2000
    return pl.pallas_call(
        _msg_body,
        grid=(_B, _N // blk),
        in_specs=[
            pl.BlockSpec((1, blk, _D), lambda b, i: (b, i, 0)),
            pl.BlockSpec((_D, _H), lambda b, i: (0, 0)),
            pl.BlockSpec((1, 1, _H), lambda b, i: (b, 0, 0)),
        ],
        out_specs=pl.BlockSpec((1, blk, _H), lambda b, i: (b, i, 0)),
        out_shape=jax.ShapeDtypeStruct((_B, _N, _H), _f32),
    )(x, w1x, c1)


def _sc_sums_body(m_hbm, nbr_hbm, seg_hbm, sums_hbm,
                  idx0, idx1, seg0, seg1, rows0, rows1, ztile, acc_sh,
                  sem0, sem1):
    b = lax.axis_index("c")
    s = lax.axis_index("s")

    z16 = jnp.zeros((16,), _f32)
    for j in range(_ZR):
        for k in range(_H // 16):
            ztile[j, pl.ds(k * 16, 16)] = z16

    @pl.loop(0, _RPS // _ZR)
    def _zinit(i):
        pltpu.sync_copy(ztile, acc_sh.at[pl.ds(s * _RPS + i * _ZR, _ZR)])

    plsc.subcore_barrier()

    ebase = b * _E + s * _EP
    bufs = ((idx0, seg0, rows0, sem0), (idx1, seg1, rows1, sem1))

    for t in range(2):
        idx_v, seg_v, rows_v, sem = bufs[t]
        pltpu.sync_copy(nbr_hbm.at[pl.ds(ebase + t * _CH, _CH)], idx_v)
        pltpu.sync_copy(seg_hbm.at[pl.ds(s * _EP + t * _CH, _CH)], seg_v)
        pltpu.async_copy(m_hbm.at[idx_v], rows_v, sem)

    @pl.loop(0, _NCHUNK, step=2)
    def _step(i):
        for t in range(2):
            idx_v, seg_v, rows_v, sem = bufs[t]
            j = i + t
            pltpu.make_async_copy(m_hbm.at[idx_v], rows_v, sem).wait()
            pltpu.sync_copy(rows_v, acc_sh.at[seg_v], add=True)
            off = ebase + (j + 2) * _CH
            pltpu.sync_copy(nbr_hbm.at[pl.ds(off, _CH)], idx_v)
            pltpu.sync_copy(seg_hbm.at[pl.ds(s * _EP + (j + 2) * _CH, _CH)],
                            seg_v)
            pltpu.async_copy(m_hbm.at[idx_v], rows_v, sem)

    for t in range(2):
        idx_v, seg_v, rows_v, sem = bufs[t]
        pltpu.make_async_copy(m_hbm.at[idx_v], rows_v, sem).wait()

    plsc.subcore_barrier()

    @pl.loop(0, _RPS // _ZR)
    def _wb(i):
        r0 = s * _RPS + i * _ZR
        pltpu.sync_copy(acc_sh.at[pl.ds(r0, _ZR)], ztile)
        pltpu.sync_copy(ztile, sums_hbm.at[b, pl.ds(r0, _ZR)])


def _sc_cnt_body(seg_hbm, cnt_hbm, seg0, seg1, ones_v, ztile, cnt_sh,
                 sem0, sem1):
    c = lax.axis_index("c")
    s = lax.axis_index("s")

    z16 = jnp.zeros((16,), _f32)
    one16 = jnp.ones((16,), _f32)
    for j in range(_ZR):
        for k in range(_H // 16):
            ztile[j, pl.ds(k * 16, 16)] = z16
    for j in range(_CH):
        for k in range(_H // 16):
            ones_v[j, pl.ds(k * 16, 16)] = one16

    @pl.loop(0, _RPS // _ZR)
    def _zinit(i):
        pltpu.sync_copy(ztile, cnt_sh.at[pl.ds(s * _RPS + i * _ZR, _ZR)])

    plsc.subcore_barrier()

    ebase = c * _E2 + s * _EP2
    bufs = ((seg0, sem0), (seg1, sem1))
    ncc = _EP2 // _CH

    for t in range(2):
        seg_v, sem = bufs[t]
        pltpu.async_copy(seg_hbm.at[pl.ds(ebase + t * _CH, _CH)], seg_v, sem)

    @pl.loop(0, ncc - 1, step=2)
    def _step(i):
        for t in range(2):
            seg_v, sem = bufs[t]
            j = i + t
            pltpu.make_async_copy(
                seg_hbm.at[pl.ds(ebase, _CH)], seg_v, sem).wait()
            pltpu.sync_copy(ones_v, cnt_sh.at[seg_v], add=True)
            pltpu.async_copy(
                seg_hbm.at[pl.ds(ebase + (j + 2) * _CH, _CH)], seg_v, sem)

    seg_v, sem = bufs[(ncc - 1) % 2]
    pltpu.make_async_copy(seg_hbm.at[pl.ds(ebase, _CH)], seg_v, sem).wait()
    pltpu.sync_copy(ones_v, cnt_sh.at[seg_v], add=True)
    seg_v, sem = bufs[ncc % 2]
    pltpu.make_async_copy(seg_hbm.at[pl.ds(ebase, _CH)], seg_v, sem).wait()

    plsc.subcore_barrier()

    @pl.loop(0, _RPS // _ZR)
    def _wb(i):
        r0 = s * _RPS + i * _ZR
        pltpu.sync_copy(cnt_sh.at[pl.ds(r0, _ZR)], ztile)
        pltpu.sync_copy(ztile, cnt_hbm.at[c, pl.ds(r0, _ZR)])


def _sc_aggregate(m_flat, nbr_flat, seg):
    mesh = plsc.VectorSubcoreMesh(core_axis_name="c", subcore_axis_name="s",
                                  num_cores=_NC, num_subcores=_NS)
    ksum = functools.partial(
        pl.kernel,
        out_type=jax.ShapeDtypeStruct((_B, _NP, _H), _f32),
        mesh=mesh,
        scratch_types=[
            pltpu.VMEM((_CH,), jnp.int32),
            pltpu.VMEM((_CH,), jnp.int32),
            pltpu.VMEM((_CH,), jnp.int32),
            pltpu.VMEM((_CH,), jnp.int32),
            pltpu.VMEM((_CH, _H), _f32),
            pltpu.VMEM((_CH, _H), _f32),
            pltpu.VMEM((_ZR, _H), _f32),
            pltpu.VMEM_SHARED((_NP, _H), _f32),
            pltpu.SemaphoreType.DMA,
            pltpu.SemaphoreType.DMA,
        ],
    )(_sc_sums_body)
    kcnt = functools.partial(
        pl.kernel,
        out_type=jax.ShapeDtypeStruct((_NC, _NP, _H), _f32),
        mesh=mesh,
        scratch_types=[
            pltpu.VMEM((_CH,), jnp.int32),
            pltpu.VMEM((_CH,), jnp.int32),
            pltpu.VMEM((_CH, _H), _f32),
            pltpu.VMEM((_ZR, _H), _f32),
            pltpu.VMEM_SHARED((_NP, _H), _f32),
            pltpu.SemaphoreType.DMA,
            pltpu.SemaphoreType.DMA,
        ],
    )(_sc_cnt_body)
    return ksum(m_flat, nbr_flat, seg), kcnt(seg)


def _upd_body(x_ref, s_ref, c2_ref, wx_ref, wh_ref, c_ref, o_ref):
    cnt = c2_ref[0][:, 0:1] + c2_ref[1][:, 0:1]
    agg = s_ref[0] * (1.0 / jnp.maximum(cnt, 1.0))
    h = (jnp.dot(x_ref[0], wx_ref[...], preferred_element_type=_f32)
         + jnp.dot(agg, wh_ref[...], preferred_element_type=_f32))
    o_ref[0] = _gelu(h + c_ref[0])


def _update(x, sums, cnt16, w2x, w2h, c2):
    blk = 2000
    return pl.pallas_call(
        _upd_body,
        grid=(_B, _N // blk),
        in_specs=[
            pl.BlockSpec((1, blk, _D), lambda b, i: (b, i, 0)),
            pl.BlockSpec((1, blk, _H), lambda b, i: (b, i, 0)),
            pl.BlockSpec((_NC, blk, _H), lambda b, i: (0, i, 0)),
            pl.BlockSpec((_D, _H), lambda b, i: (0, 0)),
            pl.BlockSpec((_H, _H), lambda b, i: (0, 0)),
            pl.BlockSpec((1, 1, _H), lambda b, i: (b, 0, 0)),
        ],
        out_specs=pl.BlockSpec((1, blk, _H), lambda b, i: (b, i, 0)),
        out_shape=jax.ShapeDtypeStruct((_B, _N, _H), _f32),
    )(x, sums, cnt16, w2x, w2h, c2)


def kernel(node_representations, edges, edge_weights, time_embed,
           gamma1, beta1, W1, b1, gamma2, beta2, W2, b2):
    x = node_representations
    inv = (1.0 + _BN_EPS) ** -0.5
    s1 = gamma1 * inv
    s2 = gamma2 * inv

    w1x = W1[:_D] * s1[:_D, None]
    c1 = (beta1 @ W1 + (time_embed * s1[_D:]) @ W1[_D:] + b1)[:, None, :]
    w2x = W2[:_D] * s2[:_D, None]
    w2h = W2[_D:_D + _H] * s2[_D:_D + _H, None]
    c2 = (beta2 @ W2 + (time_embed * s2[_D + _H:]) @ W2[_D + _H:] + b2)[:, None, :]

    msgs = _msg_table(x, w1x, c1)
    m_flat = msgs.reshape(_B * _N, _H)

    nbr = edges[:, :, 1].astype(jnp.int32)
    nbr_flat = (nbr + jnp.arange(_B, dtype=jnp.int32)[:, None] * _N).reshape(-1)
    seg = edges[0, :, 0].astype(jnp.int32)
    nbr_flat = jnp.pad(nbr_flat, (0, 2 * _CH))
    seg = jnp.pad(seg, (0, 2 * _CH))

    sums, cnt16 = _sc_aggregate(m_flat, nbr_flat, seg)

    return _update(x, sums, cnt16, w2x, w2h, c2)
